# baseline (speedup 1.0000x reference)
import functools

import jax
import jax.numpy as jnp
from jax import lax
from jax.experimental import pallas as pl
from jax.experimental.pallas import tpu as pltpu, tpu_sc as plsc

_L = 16
_NC = 2
_NS = 16
_NW = _NC * _NS
_ROWS = 4096
_IN_COLS = 512
_OUT_COLS = 256
_IN_PER_W = (_ROWS // _NW) * _IN_COLS
_OUT_PER_W = (_ROWS // _NW) * _OUT_COLS
_GROUPS = _OUT_PER_W // _L


def _sc_body(in_hbm, out_hbm, vin, vout):
    wid = lax.axis_index("s") * _NC + lax.axis_index("c")
    pltpu.sync_copy(in_hbm.at[pl.ds(wid * _IN_PER_W, _IN_PER_W)], vin)

    lane2 = 2 * lax.broadcasted_iota(jnp.int32, (_L,), 0)

    def body(j, carry):
        idx = j * (2 * _L) + lane2
        vout[pl.ds(j * _L, _L)] = plsc.load_gather(vin, [idx])
        return carry

    lax.fori_loop(0, _GROUPS, body, 0)
    pltpu.sync_copy(vout, out_hbm.at[pl.ds(wid * _OUT_PER_W, _OUT_PER_W)])


_sc_deinterleave = pl.kernel(
    _sc_body,
    out_type=jax.ShapeDtypeStruct((_ROWS * _OUT_COLS,), jnp.float32),
    mesh=plsc.VectorSubcoreMesh(core_axis_name="c", subcore_axis_name="s"),
    scratch_types=[
        pltpu.VMEM((_IN_PER_W,), jnp.float32),
        pltpu.VMEM((_OUT_PER_W,), jnp.float32),
    ],
    compiler_params=pltpu.CompilerParams(needs_layout_passes=False),
)


def kernel(firings):
    flat = firings.reshape(_ROWS * _IN_COLS)
    out = _sc_deinterleave(flat)
    return out.reshape(1, _ROWS, _OUT_COLS)


# speedup vs baseline: 1.5353x; 1.5353x over previous
import jax
import jax.numpy as jnp
from jax import lax
from jax.experimental import pallas as pl
from jax.experimental.pallas import tpu as pltpu, tpu_sc as plsc

_L = 16
_NC = 2
_NS = 16
_NW = _NC * _NS
_ROWS = 4096
_IN_COLS = 512
_OUT_COLS = 256
_ROWS_PER_W = _ROWS // _NW
_GROUPS = _ROWS_PER_W * (_OUT_COLS // _L)


def _sc_body(in_hbm, out_hbm, vin, vout):
    wid = lax.axis_index("s") * _NC + lax.axis_index("c")
    pltpu.sync_copy(in_hbm.at[pl.ds(wid * _ROWS_PER_W, _ROWS_PER_W)], vin)

    lane2 = 2 * lax.broadcasted_iota(jnp.int32, (_L,), 0)

    def body(j, carry):
        r = j // (_OUT_COLS // _L)
        g = j % (_OUT_COLS // _L)
        rows = jnp.broadcast_to(r, (_L,))
        cols = g * (2 * _L) + lane2
        vout[r, pl.ds(g * _L, _L)] = plsc.load_gather(vin, [rows, cols])
        return carry

    lax.fori_loop(0, _GROUPS, body, 0)
    pltpu.sync_copy(vout, out_hbm.at[pl.ds(wid * _ROWS_PER_W, _ROWS_PER_W)])


_sc_deinterleave = pl.kernel(
    _sc_body,
    out_type=jax.ShapeDtypeStruct((_ROWS, _OUT_COLS), jnp.float32),
    mesh=plsc.VectorSubcoreMesh(core_axis_name="c", subcore_axis_name="s"),
    scratch_types=[
        pltpu.VMEM((_ROWS_PER_W, _IN_COLS), jnp.float32),
        pltpu.VMEM((_ROWS_PER_W, _OUT_COLS), jnp.float32),
    ],
    compiler_params=pltpu.CompilerParams(needs_layout_passes=False),
)


def kernel(firings):
    out = _sc_deinterleave(firings.reshape(_ROWS, _IN_COLS))
    return out.reshape(1, _ROWS, _OUT_COLS)
